# Initial kernel scaffold; baseline (speedup 1.0000x reference)
import jax
import jax.numpy as jnp
from jax.experimental import pallas as pl


def kernel(input, target, mask):
    raise NotImplementedError("write your pallas kernel here")


# speedup vs baseline: 1.5210x; 1.5210x over previous
import jax
import jax.numpy as jnp
from jax import lax
from jax.experimental import pallas as pl
from jax.experimental.pallas import tpu as pltpu

ALPHA = 0.7
TAU_SENT = 1.0


def _loss_body(x_ref, t_ref, m_ref, ml_ref, tot_ref, acc_ref):
    i = pl.program_id(0)
    nb = pl.num_programs(0)
    x = x_ref[0]
    t = t_ref[0]
    m = m_ref[0]
    S, V = x.shape

    iota = lax.broadcasted_iota(jnp.int32, (S, V), 1)
    maxv = jnp.max(x, axis=1, keepdims=True)
    idx = jnp.min(jnp.where(x == maxv, iota, V), axis=1,
                  keepdims=True)
    tval = jnp.sum(jnp.where(iota == t, x, 0.0), axis=1,
                   keepdims=True)
    match = (idx == t).astype(jnp.float32)

    sent = jnp.exp(jnp.sum(match, axis=0, keepdims=True)
                   * (1.0 / (S * TAU_SENT)))
    mlp = jnp.sum(tval * m, axis=0, keepdims=True)
    msp = jnp.sum(m, axis=0, keepdims=True)
    outp = sent * jnp.sum(maxv * m, axis=0, keepdims=True)
    denp = sent * msp

    @pl.when(i == 0)
    def _init():
        acc_ref[...] = jnp.zeros_like(acc_ref)

    acc_ref[0:1, 0:1] += mlp
    acc_ref[1:2, 0:1] += msp
    acc_ref[2:3, 0:1] += outp
    acc_ref[3:4, 0:1] += denp

    @pl.when(i == nb - 1)
    def _finish():
        ml = -acc_ref[0:1, 0:1] / acc_ref[1:2, 0:1]
        out = -acc_ref[2:3, 0:1] / acc_ref[3:4, 0:1]
        ml_ref[...] = ml
        tot_ref[...] = ALPHA * out + (1.0 - ALPHA) * ml


def kernel(input, target, mask):
    B, S, V = input.shape
    t3 = target.astype(jnp.int32).reshape(B, S, 1)
    m3 = mask.astype(jnp.float32).reshape(B, S, 1)

    ml, tot = pl.pallas_call(
        _loss_body,
        grid=(B,),
        in_specs=[
            pl.BlockSpec((1, S, V), lambda i: (i, 0, 0)),
            pl.BlockSpec((1, S, 1), lambda i: (i, 0, 0)),
            pl.BlockSpec((1, S, 1), lambda i: (i, 0, 0)),
        ],
        out_specs=[
            pl.BlockSpec((1, 1), lambda i: (0, 0)),
            pl.BlockSpec((1, 1), lambda i: (0, 0)),
        ],
        out_shape=[
            jax.ShapeDtypeStruct((1, 1), jnp.float32),
            jax.ShapeDtypeStruct((1, 1), jnp.float32),
        ],
        scratch_shapes=[pltpu.VMEM((8, 128), jnp.float32)],
    )(input, t3, m3)
    return (ml.reshape(()), tot.reshape(()))
